# Initial kernel scaffold; baseline (speedup 1.0000x reference)
import jax
import jax.numpy as jnp
from jax.experimental import pallas as pl


def kernel(x, batch):
    raise NotImplementedError("write your pallas kernel here")


# speedup vs baseline: 4.1892x; 4.1892x over previous
import jax
import jax.numpy as jnp
from jax import lax
from jax.experimental import pallas as pl
from jax.experimental.pallas import tpu as pltpu
from jax.experimental.pallas import tpu_sc as plsc

N_NODES = 100000
D_FEAT = 128
N_GRAPHS = 1024

NC = 2
NS = 16
NW = NC * NS
LANES = 16

CHUNK = 400
NCHUNKS = N_NODES // CHUNK
G_PAD = N_GRAPHS + LANES


def _sc_body(x_hbm, batch_hbm, out_hbm, xbuf, bbuf, drop):
    wid = lax.axis_index("s") * NC + lax.axis_index("c")

    ones16 = jnp.full((LANES,), 1.0, jnp.float32)

    def init_body(g, carry):
        drop[pl.ds(g * LANES, LANES)] = ones16
        return carry

    lax.fori_loop(0, G_PAD // LANES, init_body, 0)

    def chunk_body(i, carry):
        chunk = wid + i * NW
        row0 = chunk * CHUNK
        pltpu.sync_copy(x_hbm.at[pl.ds(row0, CHUNK), :], xbuf)
        pltpu.sync_copy(batch_hbm.at[pl.ds(row0, CHUNK)], bbuf)

        def grp_body(g, gcarry):
            bvec = bbuf[pl.ds(g * LANES, LANES)]
            for j in range(LANES):
                b = bvec[j]
                dval = drop[pl.ds(b, LANES)][0]
                r = g * LANES + j
                for k in range(D_FEAT // LANES):
                    xv = xbuf[r, pl.ds(k * LANES, LANES)]
                    xbuf[r, pl.ds(k * LANES, LANES)] = xv * dval
            return gcarry

        lax.fori_loop(0, CHUNK // LANES, grp_body, 0)
        pltpu.sync_copy(xbuf, out_hbm.at[pl.ds(row0, CHUNK), :])
        return carry

    n_chunks_w = (NCHUNKS - wid + NW - 1) // NW
    lax.fori_loop(0, n_chunks_w, chunk_body, 0)


def kernel(x, batch):
    batch = batch.astype(jnp.int32)
    mesh = plsc.VectorSubcoreMesh(core_axis_name="c", subcore_axis_name="s")
    out = pl.kernel(
        _sc_body,
        out_type=jax.ShapeDtypeStruct((N_NODES, D_FEAT), jnp.float32),
        mesh=mesh,
        scratch_types=[
            pltpu.VMEM((CHUNK, D_FEAT), jnp.float32),
            pltpu.VMEM((CHUNK,), jnp.int32),
            pltpu.VMEM((G_PAD,), jnp.float32),
        ],
    )(x, batch)
    return out


# speedup vs baseline: 5.2630x; 1.2563x over previous
import jax
import jax.numpy as jnp
from jax import lax
from jax.experimental import pallas as pl
from jax.experimental.pallas import tpu as pltpu
from jax.experimental.pallas import tpu_sc as plsc

N_NODES = 100000
D_FEAT = 128
N_GRAPHS = 1024

NC = 2
NS = 16
NW = NC * NS
LANES = 16

CHUNK = 400
NCHUNKS = N_NODES // CHUNK
NITER = (NCHUNKS + NW - 1) // NW
G_PAD = N_GRAPHS + LANES


def _sc_body(x_hbm, batch_hbm, out_hbm,
             xb0, xb1, bb0, bb1, drop, isem0, isem1, osem0, osem1):
    wid = lax.axis_index("s") * NC + lax.axis_index("c")
    xbs, bbs = (xb0, xb1), (bb0, bb1)
    isems, osems = (isem0, isem1), (osem0, osem1)

    ones16 = jnp.full((LANES,), 1.0, jnp.float32)

    def init_body(g, carry):
        drop[pl.ds(g * LANES, LANES)] = ones16
        return carry

    lax.fori_loop(0, G_PAD // LANES, init_body, 0)

    def in_copies(it, p):
        c = wid + it * NW
        row0 = c * CHUNK
        return (
            pltpu.make_async_copy(x_hbm.at[pl.ds(row0, CHUNK), :], xbs[p], isems[p]),
            pltpu.make_async_copy(batch_hbm.at[pl.ds(row0, CHUNK)], bbs[p], isems[p]),
        )

    def out_copy(it, p):
        c = wid + it * NW
        row0 = c * CHUNK
        return pltpu.make_async_copy(xbs[p], out_hbm.at[pl.ds(row0, CHUNK), :], osems[p])

    def start_in(it, p):
        @pl.when(wid + it * NW < NCHUNKS)
        def _():
            cx, cb = in_copies(it, p)
            cx.start()
            cb.start()

    def wait_in(it, p):
        @pl.when(wid + it * NW < NCHUNKS)
        def _():
            cx, cb = in_copies(it, p)
            cx.wait()
            cb.wait()

    def start_out(it, p):
        @pl.when(wid + it * NW < NCHUNKS)
        def _():
            out_copy(it, p).start()

    def wait_out(it, p):
        @pl.when(wid + it * NW < NCHUNKS)
        def _():
            out_copy(it, p).wait()

    def compute(it, p):
        @pl.when(wid + it * NW < NCHUNKS)
        def _():
            xbuf, bbuf = xbs[p], bbs[p]

            def grp_body(g, gcarry):
                bvec = bbuf[pl.ds(g * LANES, LANES)]
                for j in range(LANES):
                    b = bvec[j]
                    dval = drop[pl.ds(b, LANES)][0]
                    r = g * LANES + j
                    for k in range(D_FEAT // LANES):
                        xv = xbuf[r, pl.ds(k * LANES, LANES)]
                        xbuf[r, pl.ds(k * LANES, LANES)] = xv * dval
                return gcarry

            lax.fori_loop(0, CHUNK // LANES, grp_body, 0)

    start_in(0, 0)
    for it in range(NITER):
        p = it & 1
        q = 1 - p
        if it >= 1:
            wait_out(it - 1, q)
        if it + 1 < NITER:
            start_in(it + 1, q)
        wait_in(it, p)
        compute(it, p)
        start_out(it, p)
    wait_out(NITER - 1, (NITER - 1) & 1)


def kernel(x, batch):
    batch = batch.astype(jnp.int32)
    mesh = plsc.VectorSubcoreMesh(core_axis_name="c", subcore_axis_name="s")
    out = pl.kernel(
        _sc_body,
        out_type=jax.ShapeDtypeStruct((N_NODES, D_FEAT), jnp.float32),
        mesh=mesh,
        scratch_types=[
            pltpu.VMEM((CHUNK, D_FEAT), jnp.float32),
            pltpu.VMEM((CHUNK, D_FEAT), jnp.float32),
            pltpu.VMEM((CHUNK,), jnp.int32),
            pltpu.VMEM((CHUNK,), jnp.int32),
            pltpu.VMEM((G_PAD,), jnp.float32),
            pltpu.SemaphoreType.DMA,
            pltpu.SemaphoreType.DMA,
            pltpu.SemaphoreType.DMA,
            pltpu.SemaphoreType.DMA,
        ],
    )(x, batch)
    return out


# speedup vs baseline: 5.4280x; 1.0314x over previous
import jax
import jax.numpy as jnp
from jax import lax
from jax.experimental import pallas as pl
from jax.experimental.pallas import tpu as pltpu
from jax.experimental.pallas import tpu_sc as plsc

N_NODES = 100000
D_FEAT = 128
N_GRAPHS = 1024

NC = 2
NS = 16
NW = NC * NS
LANES = 16

CHUNK = 400
HALF_A = 192
HALF_DMA = 200
NCHUNKS = N_NODES // CHUNK
NITER = (NCHUNKS + NW - 1) // NW
NGRP = CHUNK // LANES
GRP_A = HALF_A // LANES
G_PAD = N_GRAPHS + LANES


def _sc_body(x_hbm, batch_hbm, out_hbm,
             xb0, xb1, bb0, bb1, drop,
             isa0, isa1, isb0, isb1, osa0, osa1, osb0, osb1):
    wid = lax.axis_index("s") * NC + lax.axis_index("c")
    xbs, bbs = (xb0, xb1), (bb0, bb1)
    isemA, isemB = (isa0, isa1), (isb0, isb1)
    osemA, osemB = (osa0, osa1), (osb0, osb1)

    ones16 = jnp.full((LANES,), 1.0, jnp.float32)

    def init_body(g, carry):
        drop[pl.ds(g * LANES, LANES)] = ones16
        return carry

    lax.fori_loop(0, G_PAD // LANES, init_body, 0)

    def guard(it):
        return wid + it * NW < NCHUNKS

    def in_copies(it, p):
        row0 = (wid + it * NW) * CHUNK
        return (
            pltpu.make_async_copy(x_hbm.at[pl.ds(row0, HALF_DMA), :],
                                  xbs[p].at[pl.ds(0, HALF_DMA), :], isemA[p]),
            pltpu.make_async_copy(batch_hbm.at[pl.ds(row0, CHUNK)], bbs[p], isemA[p]),
            pltpu.make_async_copy(x_hbm.at[pl.ds(row0 + HALF_DMA, HALF_DMA), :],
                                  xbs[p].at[pl.ds(HALF_DMA, HALF_DMA), :], isemB[p]),
        )

    def out_copies(it, p):
        row0 = (wid + it * NW) * CHUNK
        return (
            pltpu.make_async_copy(xbs[p].at[pl.ds(0, HALF_A), :],
                                  out_hbm.at[pl.ds(row0, HALF_A), :], osemA[p]),
            pltpu.make_async_copy(xbs[p].at[pl.ds(HALF_A, CHUNK - HALF_A), :],
                                  out_hbm.at[pl.ds(row0 + HALF_A, CHUNK - HALF_A), :],
                                  osemB[p]),
        )

    def start_in(it, p):
        @pl.when(guard(it))
        def _():
            ca, cb, cc = in_copies(it, p)
            ca.start()
            cb.start()
            cc.start()

    def wait_out(it, p):
        @pl.when(guard(it))
        def _():
            oa, ob = out_copies(it, p)
            oa.wait()
            ob.wait()

    def compute_groups(xbuf, bbuf, g_lo, g_hi):
        def grp_body(g, gcarry):
            bvec = bbuf[pl.ds(g * LANES, LANES)]
            for j in range(LANES):
                b = bvec[j]
                dval = drop[pl.ds(b, LANES)][0]
                r = g * LANES + j
                for k in range(D_FEAT // LANES):
                    xv = xbuf[r, pl.ds(k * LANES, LANES)]
                    xbuf[r, pl.ds(k * LANES, LANES)] = xv * dval
            return gcarry

        lax.fori_loop(g_lo, g_hi, grp_body, 0)

    def process(it, p):
        @pl.when(guard(it))
        def _():
            ca, cb, _cc = in_copies(it, p)
            oa, ob = out_copies(it, p)
            ca.wait()
            cb.wait()
            compute_groups(xbs[p], bbs[p], 0, GRP_A)
            oa.start()
            _ca, _cb, cc = in_copies(it, p)
            cc.wait()
            compute_groups(xbs[p], bbs[p], GRP_A, NGRP)
            ob.start()

    start_in(0, 0)
    for it in range(NITER):
        p = it & 1
        q = 1 - p
        if it >= 1:
            wait_out(it - 1, q)
        if it + 1 < NITER:
            start_in(it + 1, q)
        process(it, p)
    wait_out(NITER - 1, (NITER - 1) & 1)


def kernel(x, batch):
    batch = batch.astype(jnp.int32)
    mesh = plsc.VectorSubcoreMesh(core_axis_name="c", subcore_axis_name="s")
    out = pl.kernel(
        _sc_body,
        out_type=jax.ShapeDtypeStruct((N_NODES, D_FEAT), jnp.float32),
        mesh=mesh,
        scratch_types=[
            pltpu.VMEM((CHUNK, D_FEAT), jnp.float32),
            pltpu.VMEM((CHUNK, D_FEAT), jnp.float32),
            pltpu.VMEM((CHUNK,), jnp.int32),
            pltpu.VMEM((CHUNK,), jnp.int32),
            pltpu.VMEM((G_PAD,), jnp.float32),
            pltpu.SemaphoreType.DMA,
            pltpu.SemaphoreType.DMA,
            pltpu.SemaphoreType.DMA,
            pltpu.SemaphoreType.DMA,
            pltpu.SemaphoreType.DMA,
            pltpu.SemaphoreType.DMA,
            pltpu.SemaphoreType.DMA,
            pltpu.SemaphoreType.DMA,
        ],
    )(x, batch)
    return out


# speedup vs baseline: 5.6610x; 1.0429x over previous
import jax
import jax.numpy as jnp
from jax import lax
from jax.experimental import pallas as pl
from jax.experimental.pallas import tpu as pltpu
from jax.experimental.pallas import tpu_sc as plsc

N_NODES = 100000
D_FEAT = 128
N_GRAPHS = 1024

NC = 2
NS = 16
NW = NC * NS
LANES = 16

CHUNK = 400
HALF_A = 192
HALF_DMA = 200
NCHUNKS = N_NODES // CHUNK
NITER = (NCHUNKS + NW - 1) // NW
NGRP = CHUNK // LANES
GRP_A = HALF_A // LANES
G_PAD = N_GRAPHS + LANES


def _sc_body(x_hbm, batch_hbm, out_hbm,
             xb0, xb1, bb0, bb1, drop,
             isa0, isa1, isb0, isb1, osa0, osa1, osb0, osb1):
    wid = lax.axis_index("s") * NC + lax.axis_index("c")
    xbs, bbs = (xb0, xb1), (bb0, bb1)
    isemA, isemB = (isa0, isa1), (isb0, isb1)
    osemA, osemB = (osa0, osa1), (osb0, osb1)

    ones16 = jnp.full((LANES,), 1.0, jnp.float32)

    def init_body(g, carry):
        drop[pl.ds(g * LANES, LANES)] = ones16
        return carry

    lax.fori_loop(0, G_PAD // LANES, init_body, 0)

    def guard(it):
        return wid + it * NW < NCHUNKS

    def in_copies(it, p):
        row0 = (wid + it * NW) * CHUNK
        return (
            pltpu.make_async_copy(x_hbm.at[pl.ds(row0, HALF_DMA), :],
                                  xbs[p].at[pl.ds(0, HALF_DMA), :], isemA[p]),
            pltpu.make_async_copy(batch_hbm.at[pl.ds(row0, CHUNK)], bbs[p], isemA[p]),
            pltpu.make_async_copy(x_hbm.at[pl.ds(row0 + HALF_DMA, HALF_DMA), :],
                                  xbs[p].at[pl.ds(HALF_DMA, HALF_DMA), :], isemB[p]),
        )

    def out_copies(it, p):
        row0 = (wid + it * NW) * CHUNK
        return (
            pltpu.make_async_copy(xbs[p].at[pl.ds(0, HALF_A), :],
                                  out_hbm.at[pl.ds(row0, HALF_A), :], osemA[p]),
            pltpu.make_async_copy(xbs[p].at[pl.ds(HALF_A, CHUNK - HALF_A), :],
                                  out_hbm.at[pl.ds(row0 + HALF_A, CHUNK - HALF_A), :],
                                  osemB[p]),
        )

    def start_in(it, p):
        @pl.when(guard(it))
        def _():
            ca, cb, cc = in_copies(it, p)
            ca.start()
            cb.start()
            cc.start()

    def wait_out(it, p):
        @pl.when(guard(it))
        def _():
            oa, ob = out_copies(it, p)
            oa.wait()
            ob.wait()

    def compute_groups(xbuf, bbuf, g_lo, g_hi):
        def grp_body(g, gcarry):
            bvec = bbuf[pl.ds(g * LANES, LANES)]
            b_first = bvec[0]
            b_last = bvec[LANES - 1]

            @pl.when(b_first == b_last)
            def _():
                dval = drop[pl.ds(b_first, LANES)][0]
                for j in range(LANES):
                    r = g * LANES + j
                    for k in range(D_FEAT // LANES):
                        xv = xbuf[r, pl.ds(k * LANES, LANES)]
                        xbuf[r, pl.ds(k * LANES, LANES)] = xv * dval

            @pl.when(b_first != b_last)
            def _():
                for j in range(LANES):
                    b = bvec[j]
                    dval = drop[pl.ds(b, LANES)][0]
                    r = g * LANES + j
                    for k in range(D_FEAT // LANES):
                        xv = xbuf[r, pl.ds(k * LANES, LANES)]
                        xbuf[r, pl.ds(k * LANES, LANES)] = xv * dval
            return gcarry

        lax.fori_loop(g_lo, g_hi, grp_body, 0)

    def process(it, p):
        @pl.when(guard(it))
        def _():
            ca, cb, _cc = in_copies(it, p)
            oa, ob = out_copies(it, p)
            ca.wait()
            cb.wait()
            compute_groups(xbs[p], bbs[p], 0, GRP_A)
            oa.start()
            _ca, _cb, cc = in_copies(it, p)
            cc.wait()
            compute_groups(xbs[p], bbs[p], GRP_A, NGRP)
            ob.start()

    start_in(0, 0)
    for it in range(NITER):
        p = it & 1
        q = 1 - p
        if it >= 1:
            wait_out(it - 1, q)
        if it + 1 < NITER:
            start_in(it + 1, q)
        process(it, p)
    wait_out(NITER - 1, (NITER - 1) & 1)


def kernel(x, batch):
    batch = batch.astype(jnp.int32)
    mesh = plsc.VectorSubcoreMesh(core_axis_name="c", subcore_axis_name="s")
    out = pl.kernel(
        _sc_body,
        out_type=jax.ShapeDtypeStruct((N_NODES, D_FEAT), jnp.float32),
        mesh=mesh,
        scratch_types=[
            pltpu.VMEM((CHUNK, D_FEAT), jnp.float32),
            pltpu.VMEM((CHUNK, D_FEAT), jnp.float32),
            pltpu.VMEM((CHUNK,), jnp.int32),
            pltpu.VMEM((CHUNK,), jnp.int32),
            pltpu.VMEM((G_PAD,), jnp.float32),
            pltpu.SemaphoreType.DMA,
            pltpu.SemaphoreType.DMA,
            pltpu.SemaphoreType.DMA,
            pltpu.SemaphoreType.DMA,
            pltpu.SemaphoreType.DMA,
            pltpu.SemaphoreType.DMA,
            pltpu.SemaphoreType.DMA,
            pltpu.SemaphoreType.DMA,
        ],
    )(x, batch)
    return out
